# baseline (speedup 1.0000x reference)
import functools

import jax
import jax.numpy as jnp
from jax import lax
from jax.experimental import pallas as pl
from jax.experimental.pallas import tpu as pltpu
from jax.experimental.pallas import tpu_sc as plsc

USERS = 100000
ITEMS = 100000
H = 128
B = 4096
L = 200

NC, NS = 2, 16
NW = NC * NS
BPW = B // NW
SPLITS = ((0, 112), (112, 96))
LP = 208
NBLK = LP // 16


@functools.partial(
    pl.kernel,
    out_type=jax.ShapeDtypeStruct((B, LP), jnp.float32),
    mesh=plsc.VectorSubcoreMesh(core_axis_name="c", subcore_axis_name="s"),
    scratch_types=[
        pltpu.VMEM((BPW,), jnp.int32),
        pltpu.VMEM((BPW, 2, 112), jnp.int32),
        pltpu.VMEM((BPW, H), jnp.float32),
        pltpu.VMEM((2, LP, H), jnp.float32),
        pltpu.VMEM((2, LP), jnp.float32),
        pltpu.SemaphoreType.DMA,
        pltpu.SemaphoreType.DMA,
        pltpu.SemaphoreType.DMA,
        pltpu.SemaphoreType.DMA,
        pltpu.SemaphoreType.DMA,
        pltpu.SemaphoreType.DMA,
        pltpu.SemaphoreType.DMA,
    ],
)
def _mf_sc(uid_hbm, iid_hbm, U_hbm, I_hbm, out_hbm,
           uid_v, iid_v, urows_v, rows_v, outb_v, sem_u, sem_r00, sem_r01,
           sem_r10, sem_r11, sem_o0, sem_o1):
    wid = lax.axis_index("s") * NC + lax.axis_index("c")
    base = wid * BPW
    sems = ((sem_r00, sem_r01), (sem_r10, sem_r11))
    osems = (sem_o0, sem_o1)

    pltpu.sync_copy(uid_hbm.at[pl.ds(base, BPW)], uid_v)
    pltpu.sync_copy(iid_hbm.at[pl.ds(base, BPW)], iid_v)
    pltpu.async_copy(U_hbm.at[uid_v], urows_v, sem_u).wait()

    def item_copy(b, s, half):
        off, n = SPLITS[half]
        return pltpu.make_async_copy(
            I_hbm.at[iid_v.at[b, half, pl.ds(0, n)]],
            rows_v.at[s, pl.ds(off, n)],
            sems[s][half])

    def fire(b, s):
        item_copy(b, s, 0).start()
        item_copy(b, s, 1).start()

    def drain(b, s):
        item_copy(b, s, 0).wait()
        item_copy(b, s, 1).wait()

    def out_copy(b, s):
        return pltpu.make_async_copy(
            outb_v.at[s], out_hbm.at[base + b], osems[s])

    iota16 = lax.broadcasted_iota(jnp.int32, (16,), 0)
    BITREV = tuple(int(format(j, "04b")[::-1], 2) for j in range(16))
    _DN = lax.GatherDimensionNumbers(
        offset_dims=(), collapsed_slice_dims=(0,), start_index_map=(0,))

    def _shuf(a, sft):
        idx = iota16 ^ sft
        return lax.gather(a, idx[:, None], _DN, (1,),
                          mode=lax.GatherScatterMode.PROMISE_IN_BOUNDS)

    def _combine(a, c, sft):
        mask = (iota16 & sft) == 0
        return (jnp.where(mask, a, _shuf(c, sft))
                + jnp.where(mask, _shuf(a, sft), c))

    def compute(b, s, blk_lo, blk_hi):
        u = [urows_v[b, pl.ds(k * 16, 16)] for k in range(H // 16)]

        def lblk(i, carry):
            l0 = pl.multiple_of(i * 16, 16)
            vecs = []
            for j in range(16):
                l = l0 + BITREV[j]
                acc = rows_v[s, l, pl.ds(0, 16)] * u[0]
                for k in range(1, H // 16):
                    acc = acc + rows_v[s, l, pl.ds(k * 16, 16)] * u[k]
                vecs.append(acc)
            for sft in (8, 4, 2, 1):
                vecs = [_combine(vecs[2 * p], vecs[2 * p + 1], sft)
                        for p in range(len(vecs) // 2)]
            outb_v[s, pl.ds(l0, 16)] = vecs[0]
            return carry

        lax.fori_loop(blk_lo, blk_hi, lblk, 0)

    fire(0, 0)

    def pair(g, carry):
        b0 = 2 * g
        fire(b0 + 1, 1)

        @pl.when(b0 >= 2)
        def _():
            out_copy(b0 - 2, 0).wait()

        item_copy(b0, 0, 0).wait()
        compute(b0, 0, 0, 7)
        item_copy(b0, 0, 1).wait()
        compute(b0, 0, 7, NBLK)
        out_copy(b0, 0).start()

        @pl.when(b0 + 2 < BPW)
        def _():
            fire(b0 + 2, 0)

        @pl.when(b0 >= 2)
        def _():
            out_copy(b0 - 1, 1).wait()

        item_copy(b0 + 1, 1, 0).wait()
        compute(b0 + 1, 1, 0, 7)
        item_copy(b0 + 1, 1, 1).wait()
        compute(b0 + 1, 1, 7, NBLK)
        out_copy(b0 + 1, 1).start()
        return carry

    lax.fori_loop(0, BPW // 2, pair, 0)
    out_copy(BPW - 2, 0).wait()
    out_copy(BPW - 1, 1).wait()


def kernel(userid_input, itemid_input, U, I):
    uid = userid_input.reshape(B).astype(jnp.int32)
    iid = itemid_input.reshape(B, L).astype(jnp.int32)
    iid = jnp.pad(iid, ((0, 0), (0, 224 - L))).reshape(B, 2, 112)
    return _mf_sc(uid, iid, U, I)[:, :L]


# speedup vs baseline: 6.5846x; 6.5846x over previous
import functools

import jax
import jax.numpy as jnp
from jax import lax
from jax.experimental import pallas as pl
from jax.experimental.pallas import tpu as pltpu
from jax.experimental.pallas import tpu_sc as plsc

USERS = 100000
ITEMS = 100000
H = 128
B = 4096
L = 200

NC, NS = 2, 16
NW = NC * NS
BPW = B // NW
HALF = L // 2
LP = 208
NBLK = LP // 16


@functools.partial(
    pl.kernel,
    out_type=jax.ShapeDtypeStruct((B, LP), jnp.float32),
    mesh=plsc.VectorSubcoreMesh(core_axis_name="c", subcore_axis_name="s"),
    scratch_types=[
        pltpu.VMEM((BPW,), jnp.int32),
        pltpu.VMEM((BPW, 2, HALF), jnp.int32),
        pltpu.VMEM((BPW, H), jnp.float32),
        pltpu.VMEM((2, LP, H), jnp.float32),
        pltpu.VMEM((2, LP), jnp.float32),
        pltpu.SemaphoreType.DMA,
        pltpu.SemaphoreType.DMA,
        pltpu.SemaphoreType.DMA,
        pltpu.SemaphoreType.DMA,
        pltpu.SemaphoreType.DMA,
        pltpu.SemaphoreType.DMA,
        pltpu.SemaphoreType.DMA,
    ],
)
def _mf_sc(uid_hbm, iid_hbm, U_hbm, I_hbm, out_hbm,
           uid_v, iid_v, urows_v, rows_v, outb_v, sem_u, sem_r00, sem_r01,
           sem_r10, sem_r11, sem_o0, sem_o1):
    wid = lax.axis_index("s") * NC + lax.axis_index("c")
    base = wid * BPW
    sems = ((sem_r00, sem_r01), (sem_r10, sem_r11))
    osems = (sem_o0, sem_o1)

    pltpu.sync_copy(uid_hbm.at[pl.ds(base, BPW)], uid_v)
    pltpu.sync_copy(iid_hbm.at[pl.ds(base, BPW)], iid_v)
    pltpu.async_copy(U_hbm.at[uid_v], urows_v, sem_u).wait()

    def item_copy(b, s, half):
        return pltpu.make_async_copy(
            I_hbm.at[iid_v.at[b, half]],
            rows_v.at[s, pl.ds(half * HALF, HALF)],
            sems[s][half])

    def fire(b, s):
        item_copy(b, s, 0).start()
        item_copy(b, s, 1).start()

    def drain(b, s):
        item_copy(b, s, 0).wait()
        item_copy(b, s, 1).wait()

    def out_copy(b, s):
        return pltpu.make_async_copy(
            outb_v.at[s], out_hbm.at[base + b], osems[s])

    iota16 = lax.broadcasted_iota(jnp.int32, (16,), 0)
    BITREV = tuple(int(format(j, "04b")[::-1], 2) for j in range(16))
    _DN = lax.GatherDimensionNumbers(
        offset_dims=(), collapsed_slice_dims=(0,), start_index_map=(0,))

    def _shuf(a, sft):
        idx = iota16 ^ sft
        return lax.gather(a, idx[:, None], _DN, (1,),
                          mode=lax.GatherScatterMode.PROMISE_IN_BOUNDS)

    def _combine(a, c, sft):
        mask = (iota16 & sft) == 0
        return (jnp.where(mask, a, _shuf(c, sft))
                + jnp.where(mask, _shuf(a, sft), c))

    def compute(b, s, blk_lo, blk_hi):
        u = [urows_v[b, pl.ds(k * 16, 16)] for k in range(H // 16)]

        def lblk(i, carry):
            l0 = pl.multiple_of(i * 16, 16)
            vecs = []
            for j in range(16):
                l = l0 + BITREV[j]
                acc = rows_v[s, l, pl.ds(0, 16)] * u[0]
                for k in range(1, H // 16):
                    acc = acc + rows_v[s, l, pl.ds(k * 16, 16)] * u[k]
                vecs.append(acc)
            for sft in (8, 4, 2, 1):
                vecs = [_combine(vecs[2 * p], vecs[2 * p + 1], sft)
                        for p in range(len(vecs) // 2)]
            outb_v[s, pl.ds(l0, 16)] = vecs[0]
            return carry

        lax.fori_loop(blk_lo, blk_hi, lblk, 0)

    fire(0, 0)

    def pair(g, carry):
        b0 = 2 * g
        fire(b0 + 1, 1)

        @pl.when(b0 >= 2)
        def _():
            out_copy(b0 - 2, 0).wait()

        item_copy(b0, 0, 0).wait()
        compute(b0, 0, 0, 6)
        item_copy(b0, 0, 1).wait()
        compute(b0, 0, 6, NBLK)
        out_copy(b0, 0).start()

        @pl.when(b0 + 2 < BPW)
        def _():
            fire(b0 + 2, 0)

        @pl.when(b0 >= 2)
        def _():
            out_copy(b0 - 1, 1).wait()

        item_copy(b0 + 1, 1, 0).wait()
        compute(b0 + 1, 1, 0, 6)
        item_copy(b0 + 1, 1, 1).wait()
        compute(b0 + 1, 1, 6, NBLK)
        out_copy(b0 + 1, 1).start()
        return carry

    lax.fori_loop(0, BPW // 2, pair, 0)
    out_copy(BPW - 2, 0).wait()
    out_copy(BPW - 1, 1).wait()


def kernel(userid_input, itemid_input, U, I):
    uid = userid_input.reshape(B).astype(jnp.int32)
    iid = itemid_input.reshape(B, 2, HALF).astype(jnp.int32)
    return _mf_sc(uid, iid, U, I)[:, :L]
